# initial kernel scaffold (unmeasured)
import functools

import jax
import jax.numpy as jnp
from jax import lax
from jax.experimental import pallas as pl
from jax.experimental.pallas import tpu as pltpu

N_DEV = 16
B = 64
D = 512


def kernel(x, Win0, Wout0, Win1, Wout1, Win2, Wout2):
    def body(
        x_ref, win0_ref, wout0_ref, win1_ref, wout1_ref, win2_ref, wout2_ref,
        out_ref,
        g0_ref, g1_ref, send_sems, recv0_sems, recv1_sems,
    ):
        me = lax.axis_index("i")

        barrier_sem = pltpu.get_barrier_semaphore()
        for o in range(1, N_DEV):
            tgt = lax.rem(me + o, N_DEV)
            pl.semaphore_signal(
                barrier_sem, inc=1,
                device_id=(tgt,), device_id_type=pl.DeviceIdType.MESH,
            )
        pl.semaphore_wait(barrier_sem, N_DEV - 1)

        def all_reduce(partial, g_ref, recv_sems):
            g_ref[pl.ds(me * B, B), :] = partial
            send_descs = []
            for o in range(1, N_DEV):
                tgt = lax.rem(me + o, N_DEV)
                rd = pltpu.make_async_remote_copy(
                    src_ref=g_ref.at[pl.ds(me * B, B), :],
                    dst_ref=g_ref.at[pl.ds(me * B, B), :],
                    send_sem=send_sems.at[o],
                    recv_sem=recv_sems.at[me],
                    device_id=(tgt,),
                    device_id_type=pl.DeviceIdType.MESH,
                )
                rd.start()
                send_descs.append(rd)
            for o in range(1, N_DEV):
                src = lax.rem(me + o, N_DEV)
                rd = pltpu.make_async_remote_copy(
                    src_ref=g_ref.at[pl.ds(src * B, B), :],
                    dst_ref=g_ref.at[pl.ds(src * B, B), :],
                    send_sem=send_sems.at[o],
                    recv_sem=recv_sems.at[src],
                    device_id=(src,),
                    device_id_type=pl.DeviceIdType.MESH,
                )
                rd.wait_recv()
            for rd in send_descs:
                rd.wait_send()
            allp = g_ref[...]
            return allp.reshape(N_DEV, B, D).sum(axis=0)

        def layer(xval, win_ref, wout_ref):
            h = jnp.maximum(
                jnp.dot(xval, win_ref[...], preferred_element_type=jnp.float32),
                0.0,
            )
            return jnp.dot(h, wout_ref[...], preferred_element_type=jnp.float32)

        xv = all_reduce(layer(x_ref[...], win0_ref, wout0_ref), g0_ref, recv0_sems)
        xv = all_reduce(layer(xv, win1_ref, wout1_ref), g1_ref, recv1_sems)
        xv = all_reduce(layer(xv, win2_ref, wout2_ref), g0_ref, recv0_sems)

        out_ref[...] = lax.dynamic_slice(xv, (me * (B // N_DEV), 0), (B // N_DEV, D))

        @functools.partial(pl.run_scoped, exit_sem=pltpu.SemaphoreType.REGULAR)
        def _(exit_sem):
            for o in range(1, N_DEV):
                tgt = lax.rem(me + o, N_DEV)
                pl.semaphore_signal(
                    exit_sem, inc=1,
                    device_id=(tgt,), device_id_type=pl.DeviceIdType.MESH,
                )
            pl.semaphore_wait(exit_sem, N_DEV - 1)

    return pl.pallas_call(
        body,
        out_shape=jax.ShapeDtypeStruct((B // N_DEV, D), jnp.float32),
        in_specs=[pl.BlockSpec(memory_space=pltpu.VMEM)] * 7,
        out_specs=pl.BlockSpec(memory_space=pltpu.VMEM),
        scratch_shapes=[
            pltpu.VMEM((N_DEV * B, D), jnp.float32),
            pltpu.VMEM((N_DEV * B, D), jnp.float32),
            pltpu.SemaphoreType.DMA((N_DEV,)),
            pltpu.SemaphoreType.DMA((N_DEV,)),
            pltpu.SemaphoreType.DMA((N_DEV,)),
        ],
        compiler_params=pltpu.CompilerParams(collective_id=0),
    )(x, Win0, Wout0, Win1, Wout1, Win2, Wout2)


# baseline (device time: 39518 ns/iter reference)
import jax
import jax.numpy as jnp
from jax import lax
from jax.experimental import pallas as pl
from jax.experimental.pallas import tpu as pltpu

N = 16
B = 64
D = 512
NW = 8
WR = B // NW


def kernel(x, Win0, Wout0, Win1, Wout1, Win2, Wout2):
    def body(
        x_ref, win0_ref, wout0_ref, win1_ref, wout1_ref, win2_ref, wout2_ref,
        out_ref,
        pbuf, rs0, rs1, xb0, xb1,
        rs_send, ag_send, rs_recv0, rs_recv1, ag_recv0, ag_recv1,
    ):
        me = lax.axis_index("i")
        is_owner = lax.rem(me, 2) == 0
        mw8 = (me // 2) * WR

        barrier_sem = pltpu.get_barrier_semaphore()
        for o in range(1, N):
            tgt = lax.rem(me + o, N)
            pl.semaphore_signal(
                barrier_sem, inc=1,
                device_id=(tgt,), device_id_type=pl.DeviceIdType.MESH,
            )

        def layer(xval, win_ref, wout_ref):
            h = jnp.maximum(
                jnp.dot(xval, win_ref[...], preferred_element_type=jnp.float32),
                0.0,
            )
            return jnp.dot(h, wout_ref[...], preferred_element_type=jnp.float32)

        def all_reduce(partial, rs_buf, x_buf, rs_recv, ag_recv):
            pbuf[...] = partial
            rs_rds = []
            for w in range(NW):
                own = 2 * w
                rd = pltpu.make_async_remote_copy(
                    src_ref=pbuf.at[pl.ds(WR * w, WR), :],
                    dst_ref=rs_buf.at[pl.ds(me * WR, WR), :],
                    send_sem=rs_send.at[w],
                    recv_sem=rs_recv.at[me],
                    device_id=(own,),
                    device_id_type=pl.DeviceIdType.MESH,
                )

                @pl.when(me != own)
                def _(rd=rd):
                    rd.start()

                rs_rds.append((rd, own))

            ag_rds = []
            for o in range(1, N):
                tgt = lax.rem(me + o, N)
                rd = pltpu.make_async_remote_copy(
                    src_ref=x_buf.at[pl.ds(mw8, WR), :],
                    dst_ref=x_buf.at[pl.ds(mw8, WR), :],
                    send_sem=ag_send.at[o],
                    recv_sem=ag_recv.at[me // 2],
                    device_id=(tgt,),
                    device_id_type=pl.DeviceIdType.MESH,
                )
                ag_rds.append(rd)

            @pl.when(is_owner)
            def _():
                red = pbuf[pl.ds(mw8, WR), :]
                for o in range(1, N):
                    src = lax.rem(me + o, N)
                    pltpu.make_async_remote_copy(
                        src_ref=rs_buf.at[pl.ds(src * WR, WR), :],
                        dst_ref=rs_buf.at[pl.ds(src * WR, WR), :],
                        send_sem=rs_send.at[0],
                        recv_sem=rs_recv.at[src],
                        device_id=(src,),
                        device_id_type=pl.DeviceIdType.MESH,
                    ).wait_recv()
                    red = red + rs_buf[pl.ds(src * WR, WR), :]
                x_buf[pl.ds(mw8, WR), :] = red
                for rd in ag_rds:
                    rd.start()

            for w in range(NW):
                own = 2 * w
                rd = pltpu.make_async_remote_copy(
                    src_ref=x_buf.at[pl.ds(WR * w, WR), :],
                    dst_ref=x_buf.at[pl.ds(WR * w, WR), :],
                    send_sem=rs_send.at[w],
                    recv_sem=ag_recv.at[w],
                    device_id=(own,),
                    device_id_type=pl.DeviceIdType.MESH,
                )

                @pl.when(me != own)
                def _(rd=rd):
                    rd.wait_recv()

            for rd, own in rs_rds:
                @pl.when(me != own)
                def _(rd=rd):
                    rd.wait_send()

            @pl.when(is_owner)
            def _():
                for rd in ag_rds:
                    rd.wait_send()

            return x_buf[...]

        def rs_final(partial, rs_buf, rs_recv):
            pbuf[...] = partial
            rds = []
            for o in range(1, N):
                tgt = lax.rem(me + o, N)
                tw8 = (tgt // 2) * WR
                rd = pltpu.make_async_remote_copy(
                    src_ref=pbuf.at[pl.ds(tw8, WR), :],
                    dst_ref=rs_buf.at[pl.ds(me * WR, WR), :],
                    send_sem=ag_send.at[o],
                    recv_sem=rs_recv.at[me],
                    device_id=(tgt,),
                    device_id_type=pl.DeviceIdType.MESH,
                )
                rd.start()
                rds.append(rd)
            red = pbuf[pl.ds(mw8, WR), :]
            for o in range(1, N):
                src = lax.rem(me + o, N)
                pltpu.make_async_remote_copy(
                    src_ref=rs_buf.at[pl.ds(src * WR, WR), :],
                    dst_ref=rs_buf.at[pl.ds(src * WR, WR), :],
                    send_sem=ag_send.at[o],
                    recv_sem=rs_recv.at[src],
                    device_id=(src,),
                    device_id_type=pl.DeviceIdType.MESH,
                ).wait_recv()
                red = red + rs_buf[pl.ds(src * WR, WR), :]
            for rd in rds:
                rd.wait_send()
            return red

        p0 = layer(x_ref[...], win0_ref, wout0_ref)
        pl.semaphore_wait(barrier_sem, N - 1)
        xv = all_reduce(p0, rs0, xb0, rs_recv0, ag_recv0)
        xv = all_reduce(layer(xv, win1_ref, wout1_ref),
                        rs1, xb1, rs_recv1, ag_recv1)
        red = rs_final(layer(xv, win2_ref, wout2_ref), rs0, rs_recv0)
        out_ref[...] = jnp.where(is_owner, red[0:4, :], red[4:8, :])

    return pl.pallas_call(
        body,
        out_shape=jax.ShapeDtypeStruct((B // N, D), jnp.float32),
        in_specs=[pl.BlockSpec(memory_space=pltpu.VMEM)] * 7,
        out_specs=pl.BlockSpec(memory_space=pltpu.VMEM),
        scratch_shapes=[
            pltpu.VMEM((B, D), jnp.float32),
            pltpu.VMEM((N * WR, D), jnp.float32),
            pltpu.VMEM((N * WR, D), jnp.float32),
            pltpu.VMEM((B, D), jnp.float32),
            pltpu.VMEM((B, D), jnp.float32),
            pltpu.SemaphoreType.DMA((NW,)),
            pltpu.SemaphoreType.DMA((N,)),
            pltpu.SemaphoreType.DMA((N,)),
            pltpu.SemaphoreType.DMA((N,)),
            pltpu.SemaphoreType.DMA((NW,)),
            pltpu.SemaphoreType.DMA((NW,)),
        ],
        compiler_params=pltpu.CompilerParams(collective_id=0),
    )(x, Win0, Wout0, Win1, Wout1, Win2, Wout2)


# device time: 36082 ns/iter; 1.0952x vs baseline; 1.0952x over previous
import jax
import jax.numpy as jnp
from jax import lax
from jax.experimental import pallas as pl
from jax.experimental.pallas import tpu as pltpu

N = 16
B = 64
D = 512
NW = 8
WR = B // NW


def kernel(x, Win0, Wout0, Win1, Wout1, Win2, Wout2):
    def body(
        x_ref, win0_ref, wout0_ref, win1_ref, wout1_ref, win2_ref, wout2_ref,
        out_ref,
        pbuf, rs0, rs1, xb0, xb1,
        rs_send, ag_send, rs_recv0, rs_recv1, ag_recv0, ag_recv1,
    ):
        me = lax.axis_index("i")
        is_owner = jnp.logical_and(me >= 4, me < 12)
        ow8 = lax.max(me - 4, 0) * WR
        fw8 = (me // 2) * WR

        barrier_sem = pltpu.get_barrier_semaphore()
        for o in range(1, N):
            tgt = lax.rem(me + o, N)
            pl.semaphore_signal(
                barrier_sem, inc=1,
                device_id=(tgt,), device_id_type=pl.DeviceIdType.MESH,
            )

        def layer(xval, win_ref, wout_ref):
            h = jnp.maximum(
                jnp.dot(xval, win_ref[...], preferred_element_type=jnp.float32),
                0.0,
            )
            return jnp.dot(h, wout_ref[...], preferred_element_type=jnp.float32)

        def all_reduce(partial, rs_buf, x_buf, rs_recv, ag_recv):
            pbuf[...] = partial
            rs_rds = []
            for w in range(NW):
                own = w + 4
                rd = pltpu.make_async_remote_copy(
                    src_ref=pbuf.at[pl.ds(WR * w, WR), :],
                    dst_ref=rs_buf.at[pl.ds(me * WR, WR), :],
                    send_sem=rs_send.at[w],
                    recv_sem=rs_recv.at[me],
                    device_id=(own,),
                    device_id_type=pl.DeviceIdType.MESH,
                )

                @pl.when(me != own)
                def _(rd=rd):
                    rd.start()

                rs_rds.append((rd, own))

            ag_rds = []
            for o in range(1, N):
                tgt = lax.rem(me + o, N)
                rd = pltpu.make_async_remote_copy(
                    src_ref=x_buf.at[pl.ds(ow8, WR), :],
                    dst_ref=x_buf.at[pl.ds(ow8, WR), :],
                    send_sem=ag_send.at[o],
                    recv_sem=ag_recv.at[lax.max(me - 4, 0)],
                    device_id=(tgt,),
                    device_id_type=pl.DeviceIdType.MESH,
                )
                ag_rds.append(rd)

            @pl.when(is_owner)
            def _():
                red = pbuf[pl.ds(ow8, WR), :]
                for o in range(1, N):
                    src = lax.rem(me + o, N)
                    pltpu.make_async_remote_copy(
                        src_ref=rs_buf.at[pl.ds(src * WR, WR), :],
                        dst_ref=rs_buf.at[pl.ds(src * WR, WR), :],
                        send_sem=rs_send.at[0],
                        recv_sem=rs_recv.at[src],
                        device_id=(src,),
                        device_id_type=pl.DeviceIdType.MESH,
                    ).wait_recv()
                    red = red + rs_buf[pl.ds(src * WR, WR), :]
                x_buf[pl.ds(ow8, WR), :] = red
                for rd in ag_rds:
                    rd.start()

            for w in range(NW):
                own = w + 4
                rd = pltpu.make_async_remote_copy(
                    src_ref=x_buf.at[pl.ds(WR * w, WR), :],
                    dst_ref=x_buf.at[pl.ds(WR * w, WR), :],
                    send_sem=rs_send.at[w],
                    recv_sem=ag_recv.at[w],
                    device_id=(own,),
                    device_id_type=pl.DeviceIdType.MESH,
                )

                @pl.when(me != own)
                def _(rd=rd):
                    rd.wait_recv()

            for rd, own in rs_rds:
                @pl.when(me != own)
                def _(rd=rd):
                    rd.wait_send()

            @pl.when(is_owner)
            def _():
                for rd in ag_rds:
                    rd.wait_send()

            return x_buf[...]

        def rs_final(partial, rs_buf, rs_recv):
            pbuf[...] = partial
            rds = []
            for o in range(1, N):
                tgt = lax.rem(me + o, N)
                tw8 = (tgt // 2) * WR
                rd = pltpu.make_async_remote_copy(
                    src_ref=pbuf.at[pl.ds(tw8, WR), :],
                    dst_ref=rs_buf.at[pl.ds(me * WR, WR), :],
                    send_sem=ag_send.at[o],
                    recv_sem=rs_recv.at[me],
                    device_id=(tgt,),
                    device_id_type=pl.DeviceIdType.MESH,
                )
                rd.start()
                rds.append(rd)
            red = pbuf[pl.ds(fw8, WR), :]
            for o in range(1, N):
                src = lax.rem(me + o, N)
                pltpu.make_async_remote_copy(
                    src_ref=rs_buf.at[pl.ds(src * WR, WR), :],
                    dst_ref=rs_buf.at[pl.ds(src * WR, WR), :],
                    send_sem=ag_send.at[o],
                    recv_sem=rs_recv.at[src],
                    device_id=(src,),
                    device_id_type=pl.DeviceIdType.MESH,
                ).wait_recv()
                red = red + rs_buf[pl.ds(src * WR, WR), :]
            for rd in rds:
                rd.wait_send()
            return red

        p0 = layer(x_ref[...], win0_ref, wout0_ref)
        pl.semaphore_wait(barrier_sem, N - 1)
        xv = all_reduce(p0, rs0, xb0, rs_recv0, ag_recv0)
        xv = all_reduce(layer(xv, win1_ref, wout1_ref),
                        rs1, xb1, rs_recv1, ag_recv1)
        red = rs_final(layer(xv, win2_ref, wout2_ref), rs0, rs_recv0)
        out_ref[...] = jnp.where(lax.rem(me, 2) == 0, red[0:4, :], red[4:8, :])

    return pl.pallas_call(
        body,
        out_shape=jax.ShapeDtypeStruct((B // N, D), jnp.float32),
        in_specs=[pl.BlockSpec(memory_space=pltpu.VMEM)] * 7,
        out_specs=pl.BlockSpec(memory_space=pltpu.VMEM),
        scratch_shapes=[
            pltpu.VMEM((B, D), jnp.float32),
            pltpu.VMEM((N * WR, D), jnp.float32),
            pltpu.VMEM((N * WR, D), jnp.float32),
            pltpu.VMEM((B, D), jnp.float32),
            pltpu.VMEM((B, D), jnp.float32),
            pltpu.SemaphoreType.DMA((NW,)),
            pltpu.SemaphoreType.DMA((N,)),
            pltpu.SemaphoreType.DMA((N,)),
            pltpu.SemaphoreType.DMA((N,)),
            pltpu.SemaphoreType.DMA((NW,)),
            pltpu.SemaphoreType.DMA((NW,)),
        ],
        compiler_params=pltpu.CompilerParams(collective_id=0),
    )(x, Win0, Wout0, Win1, Wout1, Win2, Wout2)
